# baseline (speedup 1.0000x reference)
import functools

import jax
import jax.numpy as jnp
from jax import lax
from jax.experimental import pallas as pl
from jax.experimental.pallas import tpu as pltpu
from jax.experimental.pallas import tpu_sc as plsc

N = 10000
NP = 10112
D = 256
E = 160000
H = D // 2
NT = 16
EPT = E // NT
B = 144
EPP = 10368
CHUNKS = EPP // B
NBUF = 2
DB = 72
DCH = EPP // DB
RPT = NP // NT
WBS = [72] * 8 + [56]

_mesh = lambda: plsc.VectorSubcoreMesh(core_axis_name="c", subcore_axis_name="s")


def _deg_body(sd3, out, idxv, gbuf, hist):
    c = lax.axis_index("c")
    s = lax.axis_index("s")
    pltpu.sync_copy(sd3.at[c, s], idxv)

    def zfill(i, _):
        for k in range(H // 16):
            gbuf[i, pl.ds(k * 16, 16)] = jnp.zeros((16,), jnp.float32)
        return 0

    lax.fori_loop(0, B, zfill, 0)
    row0 = s * RPT
    off = 0
    for w in WBS:
        pltpu.sync_copy(gbuf.at[pl.ds(0, w)], hist.at[pl.ds(row0 + off, w)])
        off += w

    def fill(i, _):
        for k in range(H // 16):
            gbuf[i, pl.ds(k * 16, 16)] = jnp.ones((16,), jnp.float32)
        return 0

    lax.fori_loop(0, B, fill, 0)
    plsc.subcore_barrier()

    def body(j, _):
        pltpu.sync_copy(gbuf.at[pl.ds(0, DB)], hist.at[idxv.at[j]], add=True)
        return 0

    lax.fori_loop(0, DCH, body, 0)
    plsc.subcore_barrier()

    off = 0
    for w in WBS:
        pltpu.sync_copy(hist.at[pl.ds(row0 + off, w)], gbuf.at[pl.ds(0, w)])
        pltpu.sync_copy(gbuf.at[pl.ds(0, w)], out.at[pl.ds(c * NP + row0 + off, w)])
        off += w


_deg_call = functools.partial(
    pl.kernel,
    _deg_body,
    out_type=jax.ShapeDtypeStruct((2 * NP, H), jnp.float32),
    scratch_types=[
        pltpu.VMEM((DCH, DB), jnp.int32),
        pltpu.VMEM((B, H), jnp.float32),
        pltpu.VMEM_SHARED((NP, H), jnp.float32),
    ],
)


def _prop_body(src2f, dstf, mi, out, gslot, dslot, buf0, buf1, agg,
               sem0, sem1, isem0, isem1, dsem0, dsem1):
    c = lax.axis_index("c")
    s = lax.axis_index("s")

    def zfill(i, _):
        for k in range(H // 16):
            buf0[i, pl.ds(k * 16, 16)] = jnp.zeros((16,), jnp.float32)
        return 0

    lax.fori_loop(0, B, zfill, 0)
    row0 = s * RPT
    off = 0
    for w in WBS:
        pltpu.sync_copy(buf0.at[pl.ds(0, w)], agg.at[pl.ds(row0 + off, w)])
        off += w
    plsc.subcore_barrier()

    bufs = [buf0, buf1]
    sems = [sem0, sem1]
    isems = [isem0, isem1]
    dsems = [dsem0, dsem1]

    def istage(j, t):
        return pltpu.make_async_copy(
            src2f.at[pl.ds((c * NT + s) * EPP + j * B, B)],
            gslot.at[pl.ds(t * B, B)], isems[t])

    def dstage(j, t, h):
        return pltpu.make_async_copy(
            dstf.at[pl.ds(s * EPP + j * B + h * (B // 2), B // 2)],
            dslot.at[2 * t + h], dsems[t])

    def gath(j, t):
        return pltpu.make_async_copy(
            mi.at[gslot.at[pl.ds(t * B, B)]], bufs[t], sems[t])

    for t in range(NBUF):
        istage(t, t).start()
        dstage(t, t, 0).start()
        dstage(t, t, 1).start()
    for t in range(NBUF):
        istage(t, t).wait()
        gath(t, t).start()

    def step(j, t, prefetch):
        gath(j, t).wait()
        if prefetch:
            istage(j + NBUF, t).start()
        dstage(j, t, 0).wait()
        dstage(j, t, 1).wait()
        pltpu.sync_copy(bufs[t].at[pl.ds(0, B // 2)],
                        agg.at[dslot.at[2 * t]], add=True)
        pltpu.sync_copy(bufs[t].at[pl.ds(B // 2, B // 2)],
                        agg.at[dslot.at[2 * t + 1]], add=True)
        if prefetch:
            dstage(j + NBUF, t, 0).start()
            dstage(j + NBUF, t, 1).start()
            istage(j + NBUF, t).wait()
            gath(j + NBUF, t).start()

    def body(kk, _):
        for t in range(NBUF):
            step(NBUF * kk + t, t, True)
        return 0

    lax.fori_loop(0, CHUNKS // NBUF - 1, body, 0)
    for t in range(NBUF):
        step(CHUNKS - NBUF + t, t, False)
    plsc.subcore_barrier()

    off = 0
    for w in WBS:
        pltpu.sync_copy(agg.at[pl.ds(row0 + off, w)], buf0.at[pl.ds(0, w)])
        pltpu.sync_copy(buf0.at[pl.ds(0, w)], out.at[pl.ds(c * NP + row0 + off, w)])
        off += w


_prop_call = functools.partial(
    pl.kernel,
    _prop_body,
    out_type=jax.ShapeDtypeStruct((2 * NP, H), jnp.float32),
    scratch_types=[
        pltpu.VMEM((NBUF * B,), jnp.int32),
        pltpu.VMEM((2 * NBUF, B // 2), jnp.int32),
        pltpu.VMEM((B, H), jnp.float32),
        pltpu.VMEM((B, H), jnp.float32),
        pltpu.VMEM_SHARED((NP, H), jnp.float32),
        pltpu.SemaphoreType.DMA,
        pltpu.SemaphoreType.DMA,
        pltpu.SemaphoreType.DMA,
        pltpu.SemaphoreType.DMA,
        pltpu.SemaphoreType.DMA,
        pltpu.SemaphoreType.DMA,
    ],
)


RB = 1000
GRID = N // RB


def _norm(deg):
    return lax.rsqrt(jnp.maximum(deg[:, 0:1], 1.0))


def _prep_body(x_ref, ds_ref, o_ref):
    o_ref[...] = (x_ref[...] * _norm(ds_ref[0])).reshape(2 * RB, H)


def _layer_body(a0_ref, a1_ref, dd_ref, ds_ref, w_ref, b_ref, o_ref):
    a = jnp.concatenate([a0_ref[0], a1_ref[0]], axis=1) * _norm(dd_ref[0])
    h = jnp.dot(a, w_ref[...], preferred_element_type=jnp.float32) + b_ref[...]
    h = jnp.maximum(h, 0.0)
    o_ref[...] = (h * _norm(ds_ref[0])).reshape(2 * RB, H)


def _final_body(a0_ref, a1_ref, dd_ref, w_ref, b_ref, o_ref):
    a = jnp.concatenate([a0_ref[0], a1_ref[0]], axis=1) * _norm(dd_ref[0])
    h = jnp.dot(a, w_ref[...], preferred_element_type=jnp.float32) + b_ref[...]
    h = h - jnp.max(h, axis=1, keepdims=True)
    e = jnp.exp(h)
    o_ref[...] = e / jnp.sum(e, axis=1, keepdims=True)


_half0_spec = pl.BlockSpec((1, RB, H), lambda r: (0, r, 0))
_deg0_spec = pl.BlockSpec((1, RB, H), lambda r: (0, r, 0))
_deg1_spec = pl.BlockSpec((1, RB, H), lambda r: (1, r, 0))
_half1_spec = pl.BlockSpec((1, RB, H), lambda r: (1, r, 0))
_w_spec = pl.BlockSpec((D, D), lambda r: (0, 0))
_b_spec = pl.BlockSpec((1, D), lambda r: (0, 0))
_mi_spec = pl.BlockSpec((2 * RB, H), lambda r: (r, 0))

_prep = pl.pallas_call(
    _prep_body,
    grid=(GRID,),
    in_specs=[pl.BlockSpec((RB, D), lambda r: (r, 0)), _deg0_spec],
    out_specs=_mi_spec,
    out_shape=jax.ShapeDtypeStruct((2 * NP, H), jnp.float32),
)

_layer = pl.pallas_call(
    _layer_body,
    grid=(GRID,),
    in_specs=[_half0_spec, _half1_spec, _deg1_spec, _deg0_spec, _w_spec, _b_spec],
    out_specs=_mi_spec,
    out_shape=jax.ShapeDtypeStruct((2 * NP, H), jnp.float32),
)

_final = pl.pallas_call(
    _final_body,
    grid=(GRID,),
    in_specs=[_half0_spec, _half1_spec, _deg1_spec, _w_spec, _b_spec],
    out_specs=pl.BlockSpec((RB, D), lambda r: (r, 0)),
    out_shape=jax.ShapeDtypeStruct((N, D), jnp.float32),
)


def kernel(x, edge_index, W1, b1, W2, b2, W3, b3):
    src = edge_index[0].astype(jnp.int32)
    dst = edge_index[1].astype(jnp.int32)
    npad = EPP - EPT
    pad_src = jnp.broadcast_to(N + jnp.arange(npad, dtype=jnp.int32) % (NP - N),
                               (NT, npad))
    pad_dst = jnp.broadcast_to(N + (jnp.arange(npad, dtype=jnp.int32) + 57) % (NP - N),
                               (NT, npad))
    srcp = jnp.concatenate([src.reshape(NT, EPT), pad_src], axis=1)
    dstp = jnp.concatenate([dst.reshape(NT, EPT), pad_dst], axis=1)
    sd3 = jnp.stack([srcp.reshape(NT, DCH, DB), dstp.reshape(NT, DCH, DB)])
    src2f = jnp.stack([2 * srcp, 2 * srcp + 1]).reshape(-1)
    dstf = dstp.reshape(-1)

    mesh = _mesh()
    deg = _deg_call(mesh=mesh)(sd3).reshape(2, NP, H)
    prop = _prop_call(mesh=mesh)

    b1r = b1.reshape(1, D)
    b2r = b2.reshape(1, D)
    b3r = b3.reshape(1, D)

    m = _prep(x, deg)
    a = prop(src2f, dstf, m).reshape(2, NP, H)
    m = _layer(a, a, deg, deg, W1, b1r)
    a = prop(src2f, dstf, m).reshape(2, NP, H)
    m = _layer(a, a, deg, deg, W2, b2r)
    a = prop(src2f, dstf, m).reshape(2, NP, H)
    return _final(a, a, deg, W3, b3r)


# speedup vs baseline: 1.0533x; 1.0533x over previous
import functools

import jax
import jax.numpy as jnp
from jax import lax
from jax.experimental import pallas as pl
from jax.experimental.pallas import tpu as pltpu
from jax.experimental.pallas import tpu_sc as plsc

N = 10000
NP = 10112
D = 256
E = 160000
H = D // 2
NT = 16
EPT = E // NT
B = 96
EPP = 10368
CHUNKS = EPP // B
NBUF = 3
DB = 72
DCH = EPP // DB
RPT = NP // NT
WBS = [72] * 8 + [56]

_mesh = lambda: plsc.VectorSubcoreMesh(core_axis_name="c", subcore_axis_name="s")


def _deg_body(sd3, out, idxv, gbuf, hist):
    c = lax.axis_index("c")
    s = lax.axis_index("s")
    pltpu.sync_copy(sd3.at[c, s], idxv)

    def zfill(i, _):
        for k in range(H // 16):
            gbuf[i, pl.ds(k * 16, 16)] = jnp.zeros((16,), jnp.float32)
        return 0

    lax.fori_loop(0, B, zfill, 0)
    row0 = s * RPT
    off = 0
    for w in WBS:
        pltpu.sync_copy(gbuf.at[pl.ds(0, w)], hist.at[pl.ds(row0 + off, w)])
        off += w

    def fill(i, _):
        for k in range(H // 16):
            gbuf[i, pl.ds(k * 16, 16)] = jnp.ones((16,), jnp.float32)
        return 0

    lax.fori_loop(0, B, fill, 0)
    plsc.subcore_barrier()

    def body(j, _):
        pltpu.sync_copy(gbuf.at[pl.ds(0, DB)], hist.at[idxv.at[j]], add=True)
        return 0

    lax.fori_loop(0, DCH, body, 0)
    plsc.subcore_barrier()

    off = 0
    for w in WBS:
        pltpu.sync_copy(hist.at[pl.ds(row0 + off, w)], gbuf.at[pl.ds(0, w)])
        pltpu.sync_copy(gbuf.at[pl.ds(0, w)], out.at[pl.ds(c * NP + row0 + off, w)])
        off += w


_deg_call = functools.partial(
    pl.kernel,
    _deg_body,
    out_type=jax.ShapeDtypeStruct((2 * NP, H), jnp.float32),
    scratch_types=[
        pltpu.VMEM((DCH, DB), jnp.int32),
        pltpu.VMEM((B, H), jnp.float32),
        pltpu.VMEM_SHARED((NP, H), jnp.float32),
    ],
)


def _prop_body(src2f, dstf, mi, out, gslot, dslot, buf0, buf1, buf2, agg,
               sem0, sem1, sem2, isem0, isem1, isem2, dsem0, dsem1, dsem2):
    c = lax.axis_index("c")
    s = lax.axis_index("s")

    def zfill(i, _):
        for k in range(H // 16):
            buf0[i, pl.ds(k * 16, 16)] = jnp.zeros((16,), jnp.float32)
        return 0

    lax.fori_loop(0, B, zfill, 0)
    row0 = s * RPT
    off = 0
    for w in WBS:
        pltpu.sync_copy(buf0.at[pl.ds(0, w)], agg.at[pl.ds(row0 + off, w)])
        off += w
    plsc.subcore_barrier()

    bufs = [buf0, buf1, buf2]
    sems = [sem0, sem1, sem2]
    isems = [isem0, isem1, isem2]
    dsems = [dsem0, dsem1, dsem2]

    def istage(j, t):
        return pltpu.make_async_copy(
            src2f.at[pl.ds((c * NT + s) * EPP + j * B, B)],
            gslot.at[pl.ds(t * B, B)], isems[t])

    def dstage(j, t):
        return pltpu.make_async_copy(
            dstf.at[pl.ds(s * EPP + j * B, B)], dslot.at[t], dsems[t])

    def gath(j, t):
        return pltpu.make_async_copy(
            mi.at[gslot.at[pl.ds(t * B, B)]], bufs[t], sems[t])

    for t in range(NBUF):
        istage(t, t).start()
        dstage(t, t).start()
    for t in range(NBUF):
        istage(t, t).wait()
        gath(t, t).start()

    def step(j, t, prefetch):
        gath(j, t).wait()
        if prefetch:
            istage(j + NBUF, t).start()
        dstage(j, t).wait()
        pltpu.sync_copy(bufs[t], agg.at[dslot.at[t]], add=True)
        if prefetch:
            dstage(j + NBUF, t).start()
            istage(j + NBUF, t).wait()
            gath(j + NBUF, t).start()

    def body(kk, _):
        for t in range(NBUF):
            step(NBUF * kk + t, t, True)
        return 0

    lax.fori_loop(0, CHUNKS // NBUF - 1, body, 0)
    for t in range(NBUF):
        step(CHUNKS - NBUF + t, t, False)
    plsc.subcore_barrier()

    off = 0
    for w in WBS:
        pltpu.sync_copy(agg.at[pl.ds(row0 + off, w)], buf0.at[pl.ds(0, w)])
        pltpu.sync_copy(buf0.at[pl.ds(0, w)], out.at[pl.ds(c * NP + row0 + off, w)])
        off += w


_prop_call = functools.partial(
    pl.kernel,
    _prop_body,
    out_type=jax.ShapeDtypeStruct((2 * NP, H), jnp.float32),
    scratch_types=[
        pltpu.VMEM((NBUF * B,), jnp.int32),
        pltpu.VMEM((NBUF, B), jnp.int32),
        pltpu.VMEM((B, H), jnp.float32),
        pltpu.VMEM((B, H), jnp.float32),
        pltpu.VMEM((B, H), jnp.float32),
        pltpu.VMEM_SHARED((NP, H), jnp.float32),
        pltpu.SemaphoreType.DMA,
        pltpu.SemaphoreType.DMA,
        pltpu.SemaphoreType.DMA,
        pltpu.SemaphoreType.DMA,
        pltpu.SemaphoreType.DMA,
        pltpu.SemaphoreType.DMA,
        pltpu.SemaphoreType.DMA,
        pltpu.SemaphoreType.DMA,
        pltpu.SemaphoreType.DMA,
    ],
)


RB = 1000
GRID = N // RB


def _norm(deg):
    return lax.rsqrt(jnp.maximum(deg[:, 0:1], 1.0))


def _prep_body(x_ref, ds_ref, o_ref):
    o_ref[...] = (x_ref[...] * _norm(ds_ref[0])).reshape(2 * RB, H)


def _layer_body(a0_ref, a1_ref, dd_ref, ds_ref, w_ref, b_ref, o_ref):
    a = jnp.concatenate([a0_ref[0], a1_ref[0]], axis=1) * _norm(dd_ref[0])
    h = jnp.dot(a, w_ref[...], preferred_element_type=jnp.float32) + b_ref[...]
    h = jnp.maximum(h, 0.0)
    o_ref[...] = (h * _norm(ds_ref[0])).reshape(2 * RB, H)


def _final_body(a0_ref, a1_ref, dd_ref, w_ref, b_ref, o_ref):
    a = jnp.concatenate([a0_ref[0], a1_ref[0]], axis=1) * _norm(dd_ref[0])
    h = jnp.dot(a, w_ref[...], preferred_element_type=jnp.float32) + b_ref[...]
    h = h - jnp.max(h, axis=1, keepdims=True)
    e = jnp.exp(h)
    o_ref[...] = e / jnp.sum(e, axis=1, keepdims=True)


_half0_spec = pl.BlockSpec((1, RB, H), lambda r: (0, r, 0))
_deg0_spec = pl.BlockSpec((1, RB, H), lambda r: (0, r, 0))
_deg1_spec = pl.BlockSpec((1, RB, H), lambda r: (1, r, 0))
_half1_spec = pl.BlockSpec((1, RB, H), lambda r: (1, r, 0))
_w_spec = pl.BlockSpec((D, D), lambda r: (0, 0))
_b_spec = pl.BlockSpec((1, D), lambda r: (0, 0))
_mi_spec = pl.BlockSpec((2 * RB, H), lambda r: (r, 0))

_prep = pl.pallas_call(
    _prep_body,
    grid=(GRID,),
    in_specs=[pl.BlockSpec((RB, D), lambda r: (r, 0)), _deg0_spec],
    out_specs=_mi_spec,
    out_shape=jax.ShapeDtypeStruct((2 * NP, H), jnp.float32),
)

_layer = pl.pallas_call(
    _layer_body,
    grid=(GRID,),
    in_specs=[_half0_spec, _half1_spec, _deg1_spec, _deg0_spec, _w_spec, _b_spec],
    out_specs=_mi_spec,
    out_shape=jax.ShapeDtypeStruct((2 * NP, H), jnp.float32),
)

_final = pl.pallas_call(
    _final_body,
    grid=(GRID,),
    in_specs=[_half0_spec, _half1_spec, _deg1_spec, _w_spec, _b_spec],
    out_specs=pl.BlockSpec((RB, D), lambda r: (r, 0)),
    out_shape=jax.ShapeDtypeStruct((N, D), jnp.float32),
)


def kernel(x, edge_index, W1, b1, W2, b2, W3, b3):
    src = edge_index[0].astype(jnp.int32)
    dst = edge_index[1].astype(jnp.int32)
    npad = EPP - EPT
    pad_src = jnp.broadcast_to(N + jnp.arange(npad, dtype=jnp.int32) % (NP - N),
                               (NT, npad))
    pad_dst = jnp.broadcast_to(N + (jnp.arange(npad, dtype=jnp.int32) + 57) % (NP - N),
                               (NT, npad))
    srcp = jnp.concatenate([src.reshape(NT, EPT), pad_src], axis=1)
    dstp = jnp.concatenate([dst.reshape(NT, EPT), pad_dst], axis=1)
    sd3 = jnp.stack([srcp.reshape(NT, DCH, DB), dstp.reshape(NT, DCH, DB)])
    src2f = jnp.stack([2 * srcp, 2 * srcp + 1]).reshape(-1)
    dstf = dstp.reshape(-1)

    mesh = _mesh()
    deg = _deg_call(mesh=mesh)(sd3).reshape(2, NP, H)
    prop = _prop_call(mesh=mesh)

    b1r = b1.reshape(1, D)
    b2r = b2.reshape(1, D)
    b3r = b3.reshape(1, D)

    m = _prep(x, deg)
    a = prop(src2f, dstf, m).reshape(2, NP, H)
    m = _layer(a, a, deg, deg, W1, b1r)
    a = prop(src2f, dstf, m).reshape(2, NP, H)
    m = _layer(a, a, deg, deg, W2, b2r)
    a = prop(src2f, dstf, m).reshape(2, NP, H)
    return _final(a, a, deg, W3, b3r)


# speedup vs baseline: 1.0649x; 1.0111x over previous
import functools

import jax
import jax.numpy as jnp
from jax import lax
from jax.experimental import pallas as pl
from jax.experimental.pallas import tpu as pltpu
from jax.experimental.pallas import tpu_sc as plsc

N = 10000
NP = 10112
D = 256
E = 160000
H = D // 2
NT = 16
EPT = E // NT
B = 72
EPP = 10368
CHUNKS = EPP // B
NBUF = 4
DB = 72
DCH = EPP // DB
RPT = NP // NT
WBS = [72] * 8 + [56]

_mesh = lambda: plsc.VectorSubcoreMesh(core_axis_name="c", subcore_axis_name="s")


def _deg_body(sd3, out, idxv, gbuf, hist):
    c = lax.axis_index("c")
    s = lax.axis_index("s")
    pltpu.sync_copy(sd3.at[c, s], idxv)

    def zfill(i, _):
        for k in range(H // 16):
            gbuf[i, pl.ds(k * 16, 16)] = jnp.zeros((16,), jnp.float32)
        return 0

    lax.fori_loop(0, DB, zfill, 0)
    row0 = s * RPT
    off = 0
    for w in WBS:
        pltpu.sync_copy(gbuf.at[pl.ds(0, w)], hist.at[pl.ds(row0 + off, w)])
        off += w

    def fill(i, _):
        for k in range(H // 16):
            gbuf[i, pl.ds(k * 16, 16)] = jnp.ones((16,), jnp.float32)
        return 0

    lax.fori_loop(0, DB, fill, 0)
    plsc.subcore_barrier()

    def body(j, _):
        pltpu.sync_copy(gbuf, hist.at[idxv.at[j]], add=True)
        return 0

    lax.fori_loop(0, DCH, body, 0)
    plsc.subcore_barrier()

    off = 0
    for w in WBS:
        pltpu.sync_copy(hist.at[pl.ds(row0 + off, w)], gbuf.at[pl.ds(0, w)])
        pltpu.sync_copy(gbuf.at[pl.ds(0, w)], out.at[pl.ds(c * NP + row0 + off, w)])
        off += w


_deg_call = functools.partial(
    pl.kernel,
    _deg_body,
    out_type=jax.ShapeDtypeStruct((2 * NP, H), jnp.float32),
    scratch_types=[
        pltpu.VMEM((DCH, DB), jnp.int32),
        pltpu.VMEM((DB, H), jnp.float32),
        pltpu.VMEM_SHARED((NP, H), jnp.float32),
    ],
)


def _prop_body(src2f, dstf, mi, out, gslot, dslot, buf0, buf1, buf2, buf3, agg,
               sem0, sem1, sem2, sem3, isem0, isem1, isem2, isem3,
               dsem0, dsem1, dsem2, dsem3):
    c = lax.axis_index("c")
    s = lax.axis_index("s")

    def zfill(i, _):
        for k in range(H // 16):
            buf0[i, pl.ds(k * 16, 16)] = jnp.zeros((16,), jnp.float32)
        return 0

    lax.fori_loop(0, B, zfill, 0)
    row0 = s * RPT
    off = 0
    for w in WBS:
        pltpu.sync_copy(buf0.at[pl.ds(0, w)], agg.at[pl.ds(row0 + off, w)])
        off += w
    plsc.subcore_barrier()

    bufs = [buf0, buf1, buf2, buf3]
    sems = [sem0, sem1, sem2, sem3]
    isems = [isem0, isem1, isem2, isem3]
    dsems = [dsem0, dsem1, dsem2, dsem3]

    def istage(j, t):
        return pltpu.make_async_copy(
            src2f.at[pl.ds((c * NT + s) * EPP + j * B, B)],
            gslot.at[pl.ds(t * B, B)], isems[t])

    def dstage(j, t):
        return pltpu.make_async_copy(
            dstf.at[pl.ds(s * EPP + j * B, B)], dslot.at[t], dsems[t])

    def gath(j, t):
        return pltpu.make_async_copy(
            mi.at[gslot.at[pl.ds(t * B, B)]], bufs[t], sems[t])

    for t in range(NBUF):
        istage(t, t).start()
        dstage(t, t).start()
    for t in range(NBUF):
        istage(t, t).wait()
        gath(t, t).start()

    def step(j, t, prefetch):
        gath(j, t).wait()
        if prefetch:
            istage(j + NBUF, t).start()
        dstage(j, t).wait()
        pltpu.sync_copy(bufs[t], agg.at[dslot.at[t]], add=True)
        if prefetch:
            dstage(j + NBUF, t).start()
            istage(j + NBUF, t).wait()
            gath(j + NBUF, t).start()

    def body(kk, _):
        for t in range(NBUF):
            step(NBUF * kk + t, t, True)
        return 0

    lax.fori_loop(0, CHUNKS // NBUF - 1, body, 0)
    for t in range(NBUF):
        step(CHUNKS - NBUF + t, t, False)
    plsc.subcore_barrier()

    off = 0
    for w in WBS:
        pltpu.sync_copy(agg.at[pl.ds(row0 + off, w)], buf0.at[pl.ds(0, w)])
        pltpu.sync_copy(buf0.at[pl.ds(0, w)], out.at[pl.ds(c * NP + row0 + off, w)])
        off += w


_prop_call = functools.partial(
    pl.kernel,
    _prop_body,
    out_type=jax.ShapeDtypeStruct((2 * NP, H), jnp.float32),
    scratch_types=[
        pltpu.VMEM((NBUF * B,), jnp.int32),
        pltpu.VMEM((NBUF, B), jnp.int32),
        pltpu.VMEM((B, H), jnp.float32),
        pltpu.VMEM((B, H), jnp.float32),
        pltpu.VMEM((B, H), jnp.float32),
        pltpu.VMEM((B, H), jnp.float32),
        pltpu.VMEM_SHARED((NP, H), jnp.float32),
    ] + [pltpu.SemaphoreType.DMA] * 12,
)


RB = 1000
GRID = N // RB


def _norm(deg):
    return lax.rsqrt(jnp.maximum(deg[:, 0:1], 1.0))


def _prep_body(x_ref, ds_ref, o_ref):
    o_ref[...] = (x_ref[...] * _norm(ds_ref[0])).reshape(2 * RB, H)


def _layer_body(a0_ref, a1_ref, dd_ref, ds_ref, w_ref, b_ref, o_ref):
    a = jnp.concatenate([a0_ref[0], a1_ref[0]], axis=1) * _norm(dd_ref[0])
    h = jnp.dot(a, w_ref[...], preferred_element_type=jnp.float32) + b_ref[...]
    h = jnp.maximum(h, 0.0)
    o_ref[...] = (h * _norm(ds_ref[0])).reshape(2 * RB, H)


def _final_body(a0_ref, a1_ref, dd_ref, w_ref, b_ref, o_ref):
    a = jnp.concatenate([a0_ref[0], a1_ref[0]], axis=1) * _norm(dd_ref[0])
    h = jnp.dot(a, w_ref[...], preferred_element_type=jnp.float32) + b_ref[...]
    h = h - jnp.max(h, axis=1, keepdims=True)
    e = jnp.exp(h)
    o_ref[...] = e / jnp.sum(e, axis=1, keepdims=True)


_half0_spec = pl.BlockSpec((1, RB, H), lambda r: (0, r, 0))
_deg0_spec = pl.BlockSpec((1, RB, H), lambda r: (0, r, 0))
_deg1_spec = pl.BlockSpec((1, RB, H), lambda r: (1, r, 0))
_half1_spec = pl.BlockSpec((1, RB, H), lambda r: (1, r, 0))
_w_spec = pl.BlockSpec((D, D), lambda r: (0, 0))
_b_spec = pl.BlockSpec((1, D), lambda r: (0, 0))
_mi_spec = pl.BlockSpec((2 * RB, H), lambda r: (r, 0))

_prep = pl.pallas_call(
    _prep_body,
    grid=(GRID,),
    in_specs=[pl.BlockSpec((RB, D), lambda r: (r, 0)), _deg0_spec],
    out_specs=_mi_spec,
    out_shape=jax.ShapeDtypeStruct((2 * NP, H), jnp.float32),
)

_layer = pl.pallas_call(
    _layer_body,
    grid=(GRID,),
    in_specs=[_half0_spec, _half1_spec, _deg1_spec, _deg0_spec, _w_spec, _b_spec],
    out_specs=_mi_spec,
    out_shape=jax.ShapeDtypeStruct((2 * NP, H), jnp.float32),
)

_final = pl.pallas_call(
    _final_body,
    grid=(GRID,),
    in_specs=[_half0_spec, _half1_spec, _deg1_spec, _w_spec, _b_spec],
    out_specs=pl.BlockSpec((RB, D), lambda r: (r, 0)),
    out_shape=jax.ShapeDtypeStruct((N, D), jnp.float32),
)


def kernel(x, edge_index, W1, b1, W2, b2, W3, b3):
    src = edge_index[0].astype(jnp.int32)
    dst = edge_index[1].astype(jnp.int32)
    npad = EPP - EPT
    pad_src = jnp.broadcast_to(N + jnp.arange(npad, dtype=jnp.int32) % (NP - N),
                               (NT, npad))
    pad_dst = jnp.broadcast_to(N + (jnp.arange(npad, dtype=jnp.int32) + 57) % (NP - N),
                               (NT, npad))
    srcp = jnp.concatenate([src.reshape(NT, EPT), pad_src], axis=1)
    dstp = jnp.concatenate([dst.reshape(NT, EPT), pad_dst], axis=1)
    sd3 = jnp.stack([srcp.reshape(NT, DCH, DB), dstp.reshape(NT, DCH, DB)])
    src2f = jnp.stack([2 * srcp, 2 * srcp + 1]).reshape(-1)
    dstf = dstp.reshape(-1)

    mesh = _mesh()
    deg = _deg_call(mesh=mesh)(sd3).reshape(2, NP, H)
    prop = _prop_call(mesh=mesh)

    b1r = b1.reshape(1, D)
    b2r = b2.reshape(1, D)
    b3r = b3.reshape(1, D)

    m = _prep(x, deg)
    a = prop(src2f, dstf, m).reshape(2, NP, H)
    m = _layer(a, a, deg, deg, W1, b1r)
    a = prop(src2f, dstf, m).reshape(2, NP, H)
    m = _layer(a, a, deg, deg, W2, b2r)
    a = prop(src2f, dstf, m).reshape(2, NP, H)
    return _final(a, a, deg, W3, b3r)


# speedup vs baseline: 1.0818x; 1.0159x over previous
import functools

import jax
import jax.numpy as jnp
from jax import lax
from jax.experimental import pallas as pl
from jax.experimental.pallas import tpu as pltpu
from jax.experimental.pallas import tpu_sc as plsc

N = 10000
NP = 10112
D = 256
E = 160000
H = D // 2
NT = 16
EPT = E // NT
B = 64
EPP = 10240
CHUNKS = EPP // B
NBUF = 5
DB = 64
DCH = EPP // DB
RPT = NP // NT
WBS = [64] * 9 + [56]

_mesh = lambda: plsc.VectorSubcoreMesh(core_axis_name="c", subcore_axis_name="s")


def _deg_body(sd3, out, idxv, gbuf, hist):
    c = lax.axis_index("c")
    s = lax.axis_index("s")
    pltpu.sync_copy(sd3.at[c, s], idxv)

    def zfill(i, _):
        for k in range(H // 16):
            gbuf[i, pl.ds(k * 16, 16)] = jnp.zeros((16,), jnp.float32)
        return 0

    lax.fori_loop(0, DB, zfill, 0)
    row0 = s * RPT
    off = 0
    for w in WBS:
        pltpu.sync_copy(gbuf.at[pl.ds(0, w)], hist.at[pl.ds(row0 + off, w)])
        off += w

    def fill(i, _):
        for k in range(H // 16):
            gbuf[i, pl.ds(k * 16, 16)] = jnp.ones((16,), jnp.float32)
        return 0

    lax.fori_loop(0, DB, fill, 0)
    plsc.subcore_barrier()

    def body(j, _):
        pltpu.sync_copy(gbuf, hist.at[idxv.at[j]], add=True)
        return 0

    lax.fori_loop(0, DCH, body, 0)
    plsc.subcore_barrier()

    off = 0
    for w in WBS:
        pltpu.sync_copy(hist.at[pl.ds(row0 + off, w)], gbuf.at[pl.ds(0, w)])
        pltpu.sync_copy(gbuf.at[pl.ds(0, w)], out.at[pl.ds(c * NP + row0 + off, w)])
        off += w


_deg_call = functools.partial(
    pl.kernel,
    _deg_body,
    out_type=jax.ShapeDtypeStruct((2 * NP, H), jnp.float32),
    scratch_types=[
        pltpu.VMEM((DCH, DB), jnp.int32),
        pltpu.VMEM((DB, H), jnp.float32),
        pltpu.VMEM_SHARED((NP, H), jnp.float32),
    ],
)


def _prop_body(src2f, dstf, mi, out, gslot, dslot,
               buf0, buf1, buf2, buf3, buf4, agg,
               sem0, sem1, sem2, sem3, sem4,
               isem0, isem1, isem2, isem3, isem4,
               dsem0, dsem1, dsem2, dsem3, dsem4):
    c = lax.axis_index("c")
    s = lax.axis_index("s")

    def zfill(i, _):
        for k in range(H // 16):
            buf0[i, pl.ds(k * 16, 16)] = jnp.zeros((16,), jnp.float32)
        return 0

    lax.fori_loop(0, B, zfill, 0)
    row0 = s * RPT
    off = 0
    for w in WBS:
        pltpu.sync_copy(buf0.at[pl.ds(0, w)], agg.at[pl.ds(row0 + off, w)])
        off += w
    plsc.subcore_barrier()

    bufs = [buf0, buf1, buf2, buf3, buf4]
    sems = [sem0, sem1, sem2, sem3, sem4]
    isems = [isem0, isem1, isem2, isem3, isem4]
    dsems = [dsem0, dsem1, dsem2, dsem3, dsem4]

    def istage(j, t):
        return pltpu.make_async_copy(
            src2f.at[pl.ds((c * NT + s) * EPP + j * B, B)],
            gslot.at[pl.ds(t * B, B)], isems[t])

    def dstage(j, t):
        return pltpu.make_async_copy(
            dstf.at[pl.ds(s * EPP + j * B, B)], dslot.at[t], dsems[t])

    def gath(j, t):
        return pltpu.make_async_copy(
            mi.at[gslot.at[pl.ds(t * B, B)]], bufs[t], sems[t])

    for t in range(NBUF):
        istage(t, t).start()
        dstage(t, t).start()
    for t in range(NBUF):
        istage(t, t).wait()
        gath(t, t).start()

    def step(j, t, prefetch):
        gath(j, t).wait()
        if prefetch:
            istage(j + NBUF, t).start()
        dstage(j, t).wait()
        pltpu.sync_copy(bufs[t], agg.at[dslot.at[t]], add=True)
        if prefetch:
            dstage(j + NBUF, t).start()
            istage(j + NBUF, t).wait()
            gath(j + NBUF, t).start()

    def body(kk, _):
        for t in range(NBUF):
            step(NBUF * kk + t, t, True)
        return 0

    lax.fori_loop(0, CHUNKS // NBUF - 1, body, 0)
    for t in range(NBUF):
        step(CHUNKS - NBUF + t, t, False)
    plsc.subcore_barrier()

    off = 0
    for w in WBS:
        pltpu.sync_copy(agg.at[pl.ds(row0 + off, w)], buf0.at[pl.ds(0, w)])
        pltpu.sync_copy(buf0.at[pl.ds(0, w)], out.at[pl.ds(c * NP + row0 + off, w)])
        off += w


_prop_call = functools.partial(
    pl.kernel,
    _prop_body,
    out_type=jax.ShapeDtypeStruct((2 * NP, H), jnp.float32),
    scratch_types=[
        pltpu.VMEM((NBUF * B,), jnp.int32),
        pltpu.VMEM((NBUF, B), jnp.int32),
        pltpu.VMEM((B, H), jnp.float32),
        pltpu.VMEM((B, H), jnp.float32),
        pltpu.VMEM((B, H), jnp.float32),
        pltpu.VMEM((B, H), jnp.float32),
        pltpu.VMEM((B, H), jnp.float32),
        pltpu.VMEM_SHARED((NP, H), jnp.float32),
    ] + [pltpu.SemaphoreType.DMA] * 15,
)


RB = 1000
GRID = N // RB


def _norm(deg):
    return lax.rsqrt(jnp.maximum(deg[:, 0:1], 1.0))


def _prep_body(x_ref, ds_ref, o_ref):
    o_ref[...] = (x_ref[...] * _norm(ds_ref[0])).reshape(2 * RB, H)


def _layer_body(a0_ref, a1_ref, dd_ref, ds_ref, w_ref, b_ref, o_ref):
    a = jnp.concatenate([a0_ref[0], a1_ref[0]], axis=1) * _norm(dd_ref[0])
    h = jnp.dot(a, w_ref[...], preferred_element_type=jnp.float32) + b_ref[...]
    h = jnp.maximum(h, 0.0)
    o_ref[...] = (h * _norm(ds_ref[0])).reshape(2 * RB, H)


def _final_body(a0_ref, a1_ref, dd_ref, w_ref, b_ref, o_ref):
    a = jnp.concatenate([a0_ref[0], a1_ref[0]], axis=1) * _norm(dd_ref[0])
    h = jnp.dot(a, w_ref[...], preferred_element_type=jnp.float32) + b_ref[...]
    h = h - jnp.max(h, axis=1, keepdims=True)
    e = jnp.exp(h)
    o_ref[...] = e / jnp.sum(e, axis=1, keepdims=True)


_half0_spec = pl.BlockSpec((1, RB, H), lambda r: (0, r, 0))
_deg0_spec = pl.BlockSpec((1, RB, H), lambda r: (0, r, 0))
_deg1_spec = pl.BlockSpec((1, RB, H), lambda r: (1, r, 0))
_half1_spec = pl.BlockSpec((1, RB, H), lambda r: (1, r, 0))
_w_spec = pl.BlockSpec((D, D), lambda r: (0, 0))
_b_spec = pl.BlockSpec((1, D), lambda r: (0, 0))
_mi_spec = pl.BlockSpec((2 * RB, H), lambda r: (r, 0))

_prep = pl.pallas_call(
    _prep_body,
    grid=(GRID,),
    in_specs=[pl.BlockSpec((RB, D), lambda r: (r, 0)), _deg0_spec],
    out_specs=_mi_spec,
    out_shape=jax.ShapeDtypeStruct((2 * NP, H), jnp.float32),
)

_layer = pl.pallas_call(
    _layer_body,
    grid=(GRID,),
    in_specs=[_half0_spec, _half1_spec, _deg1_spec, _deg0_spec, _w_spec, _b_spec],
    out_specs=_mi_spec,
    out_shape=jax.ShapeDtypeStruct((2 * NP, H), jnp.float32),
)

_final = pl.pallas_call(
    _final_body,
    grid=(GRID,),
    in_specs=[_half0_spec, _half1_spec, _deg1_spec, _w_spec, _b_spec],
    out_specs=pl.BlockSpec((RB, D), lambda r: (r, 0)),
    out_shape=jax.ShapeDtypeStruct((N, D), jnp.float32),
)


def kernel(x, edge_index, W1, b1, W2, b2, W3, b3):
    src = edge_index[0].astype(jnp.int32)
    dst = edge_index[1].astype(jnp.int32)
    npad = EPP - EPT
    pad_src = jnp.broadcast_to(N + jnp.arange(npad, dtype=jnp.int32) % (NP - N),
                               (NT, npad))
    pad_dst = jnp.broadcast_to(N + (jnp.arange(npad, dtype=jnp.int32) + 57) % (NP - N),
                               (NT, npad))
    srcp = jnp.concatenate([src.reshape(NT, EPT), pad_src], axis=1)
    dstp = jnp.concatenate([dst.reshape(NT, EPT), pad_dst], axis=1)
    sd3 = jnp.stack([srcp.reshape(NT, DCH, DB), dstp.reshape(NT, DCH, DB)])
    src2f = jnp.stack([2 * srcp, 2 * srcp + 1]).reshape(-1)
    dstf = dstp.reshape(-1)

    mesh = _mesh()
    deg = _deg_call(mesh=mesh)(sd3).reshape(2, NP, H)
    prop = _prop_call(mesh=mesh)

    b1r = b1.reshape(1, D)
    b2r = b2.reshape(1, D)
    b3r = b3.reshape(1, D)

    m = _prep(x, deg)
    a = prop(src2f, dstf, m).reshape(2, NP, H)
    m = _layer(a, a, deg, deg, W1, b1r)
    a = prop(src2f, dstf, m).reshape(2, NP, H)
    m = _layer(a, a, deg, deg, W2, b2r)
    a = prop(src2f, dstf, m).reshape(2, NP, H)
    return _final(a, a, deg, W3, b3r)
